# Initial kernel scaffold; baseline (speedup 1.0000x reference)
import jax
import jax.numpy as jnp
from jax.experimental import pallas as pl


def kernel(input, W0, b0, W1, b1, W2, b2, W3, b3, knn0, knn1, knn2, knn3, th0, th1, fcW, fcb):
    raise NotImplementedError("write your pallas kernel here")


# speedup vs baseline: 6.4288x; 6.4288x over previous
import functools

import jax
import jax.numpy as jnp
from jax import lax
from jax.experimental import pallas as pl
from jax.experimental.pallas import tpu as pltpu
from jax.experimental.pallas import tpu_sc as plsc

T, ALPHA, BETA = 20, 0.9, 0.8
B, K, IN = 16, 16, 10000
D0, D1, D2, D3 = 5000, 2500, 1250, 625
P0, P1, P2, P3 = 5120, 2560, 1280, 768
NW = 32
N0, N1, N2, N3 = P0 // NW, P1 // NW, P2 // NW, P3 // NW
C0, C1, C2, C3 = N0 * K // 128, N1 * K // 128, N2 * K // 128, N3 * K // 128


def _mesh():
    return plsc.VectorSubcoreMesh(core_axis_name="c", subcore_axis_name="s")


_CP = pltpu.CompilerParams(use_tc_tiling_on_sc=False)


def _wid():
    return lax.axis_index("c") * 16 + lax.axis_index("s")


def _weighted_sum(xg_v, wb_v, b_v, j):
    acc = b_v[j]
    for k in range(K):
        r = j * K + k
        acc = acc + wb_v[r] * xg_v[r]
    return acc


def _spiking_kernel(N, C, stride_out):
    def body(tbl_h, idx_h, wb_h, b_h, th_h, out_h, xg_v, wb_v, b_v, th_v,
             idx_v, syn_v, mem_v, h_v, sem):
        w = _wid()
        pltpu.sync_copy(wb_h.at[pl.ds(w * (N * K), N * K)], wb_v)
        pltpu.sync_copy(b_h.at[pl.ds(w * N, N)], b_v)
        pltpu.sync_copy(th_h.at[pl.ds(w * N, N)], th_v)

        @pl.loop(0, N)
        def _(j):
            z = jnp.zeros((B,), jnp.float32)
            syn_v[j] = z
            mem_v[j] = z

        @pl.loop(0, T)
        def _(t):
            pltpu.sync_copy(idx_h.at[w, t], idx_v)
            handles = [
                pltpu.async_copy(tbl_h.at[idx_v.at[c]],
                                 xg_v.at[pl.ds(c * 128, 128)], sem)
                for c in range(C)
            ]
            for h in handles:
                h.wait()

            @pl.loop(0, N)
            def _(j):
                acc = _weighted_sum(xg_v, wb_v, b_v, j)
                th = th_v[j]
                old_mem = mem_v[j]
                reset = jnp.where(old_mem - th > 0, th, 0.0)
                syn = ALPHA * syn_v[j] + acc
                mem = BETA * old_mem + syn - reset
                syn_v[j] = syn
                mem_v[j] = mem
                h_v[j] = jnp.where(mem - th > 0, 1.0, 0.0)

            if stride_out is not None:
                pltpu.sync_copy(h_v, out_h.at[pl.ds(t * stride_out + w * N, N)])

        if stride_out is None:
            pltpu.sync_copy(mem_v, out_h.at[pl.ds(w * N, N)])

    return body


def _relu_kernel(N, C):
    def body(tbl_h, idx_h, wb_h, b_h, out_h, xg_v, wb_v, b_v, idx_v, o_v, sem):
        w = _wid()
        pltpu.sync_copy(wb_h.at[pl.ds(w * (N * K), N * K)], wb_v)
        pltpu.sync_copy(b_h.at[pl.ds(w * N, N)], b_v)
        pltpu.sync_copy(idx_h.at[w], idx_v)
        handles = [
            pltpu.async_copy(tbl_h.at[idx_v.at[c]],
                             xg_v.at[pl.ds(c * 128, 128)], sem)
            for c in range(C)
        ]
        for h in handles:
            h.wait()

        @pl.loop(0, N)
        def _(j):
            acc = _weighted_sum(xg_v, wb_v, b_v, j)
            o_v[j] = jnp.maximum(acc, 0.0)

        pltpu.sync_copy(o_v, out_h.at[pl.ds(w * N, N)])

    return body


def _fc_kernel(x3_h, fcw_h, fcb_h, out_h, x3_v, fcw_v, acc_v, sem):
    w = _wid()

    @pl.when(w == 0)
    def _():
        pltpu.sync_copy(x3_h, x3_v)
        pltpu.sync_copy(fcw_h, fcw_v)
        pltpu.sync_copy(fcb_h, acc_v)
        for o in range(2):
            @pl.loop(0, P3)
            def _(d):
                acc_v[o] = acc_v[o] + fcw_v[o * P3 + d] * x3_v[d]
        pltpu.sync_copy(acc_v, out_h)


def _pad_rows(a, P):
    pad = P - a.shape[0]
    if pad == 0:
        return a
    return jnp.concatenate([a, jnp.zeros((pad,) + a.shape[1:], a.dtype)], axis=0)


def _prep(knn, W, bvec, P, stride_t=None):
    knnp = _pad_rows(knn.astype(jnp.int32), P)
    Wp = _pad_rows(W, P)
    flat = knnp.reshape(-1)
    if stride_t is not None:
        idx = flat[None, :] + (jnp.arange(T, dtype=jnp.int32) * stride_t)[:, None]
        idx = idx.reshape(T, NW, -1, 128).transpose(1, 0, 2, 3)
    else:
        idx = flat.reshape(NW, -1, 128)
    wb = jnp.broadcast_to(Wp.reshape(-1, 1), (P * K, B)).astype(jnp.float32)
    bb = jnp.broadcast_to(_pad_rows(bvec.reshape(-1, 1), P), (P, B)).astype(jnp.float32)
    return idx, wb, bb


def kernel(input, W0, b0, W1, b1, W2, b2, W3, b3, knn0, knn1, knn2, knn3,
           th0, th1, fcW, fcb):
    f32 = jnp.float32
    xT = input.transpose(1, 2, 0).reshape(T * IN, B)

    idx0, wb0, b0b = _prep(knn0, W0, b0, P0, stride_t=IN)
    idx1, wb1, b1b = _prep(knn1, W1, b1, P1, stride_t=P0)
    idx2, wb2, b2b = _prep(knn2, W2, b2, P2)
    idx3, wb3, b3b = _prep(knn3, W3, b3, P3)
    th0b = jnp.broadcast_to(_pad_rows(th0.reshape(-1, 1), P0), (P0, B)).astype(f32)
    th1b = jnp.broadcast_to(_pad_rows(th1.reshape(-1, 1), P1), (P1, B)).astype(f32)
    fcWb = jnp.broadcast_to(
        _pad_rows(fcW.T, P3).T.reshape(-1, 1), (2 * P3, B)).astype(f32)
    fcbb = jnp.broadcast_to(fcb.reshape(-1, 1), (2, B)).astype(f32)

    kA = functools.partial(
        pl.kernel, compiler_params=_CP, out_type=jax.ShapeDtypeStruct((T * P0, B), f32), mesh=_mesh(),
        scratch_types=[
            pltpu.VMEM((N0 * K, B), f32), pltpu.VMEM((N0 * K, B), f32),
            pltpu.VMEM((N0, B), f32), pltpu.VMEM((N0, B), f32),
            pltpu.VMEM((C0, 128), jnp.int32),
            pltpu.VMEM((N0, B), f32), pltpu.VMEM((N0, B), f32),
            pltpu.VMEM((N0, B), f32), pltpu.SemaphoreType.DMA,
        ])(_spiking_kernel(N0, C0, P0))
    h0 = kA(xT, idx0, wb0, b0b, th0b)

    kB = functools.partial(
        pl.kernel, compiler_params=_CP, out_type=jax.ShapeDtypeStruct((P1, B), f32), mesh=_mesh(),
        scratch_types=[
            pltpu.VMEM((N1 * K, B), f32), pltpu.VMEM((N1 * K, B), f32),
            pltpu.VMEM((N1, B), f32), pltpu.VMEM((N1, B), f32),
            pltpu.VMEM((C1, 128), jnp.int32),
            pltpu.VMEM((N1, B), f32), pltpu.VMEM((N1, B), f32),
            pltpu.VMEM((N1, B), f32), pltpu.SemaphoreType.DMA,
        ])(_spiking_kernel(N1, C1, None))
    m1 = kB(h0, idx1, wb1, b1b, th1b)

    kC = functools.partial(
        pl.kernel, compiler_params=_CP, out_type=jax.ShapeDtypeStruct((P2, B), f32), mesh=_mesh(),
        scratch_types=[
            pltpu.VMEM((N2 * K, B), f32), pltpu.VMEM((N2 * K, B), f32),
            pltpu.VMEM((N2, B), f32), pltpu.VMEM((C2, 128), jnp.int32),
            pltpu.VMEM((N2, B), f32), pltpu.SemaphoreType.DMA,
        ])(_relu_kernel(N2, C2))
    x2 = kC(m1, idx2, wb2, b2b)

    kD = functools.partial(
        pl.kernel, compiler_params=_CP, out_type=jax.ShapeDtypeStruct((P3, B), f32), mesh=_mesh(),
        scratch_types=[
            pltpu.VMEM((N3 * K, B), f32), pltpu.VMEM((N3 * K, B), f32),
            pltpu.VMEM((N3, B), f32), pltpu.VMEM((C3, 128), jnp.int32),
            pltpu.VMEM((N3, B), f32), pltpu.SemaphoreType.DMA,
        ])(_relu_kernel(N3, C3))
    x3 = kD(x2, idx3, wb3, b3b)

    kE = functools.partial(
        pl.kernel, compiler_params=_CP, out_type=jax.ShapeDtypeStruct((2, B), f32), mesh=_mesh(),
        scratch_types=[
            pltpu.VMEM((P3, B), f32), pltpu.VMEM((2 * P3, B), f32),
            pltpu.VMEM((2, B), f32), pltpu.SemaphoreType.DMA,
        ])(_fc_kernel)
    angle = kE(x3, fcWb, fcbb)

    return angle.T


# speedup vs baseline: 7.3097x; 1.1370x over previous
import functools

import jax
import jax.numpy as jnp
from jax import lax
from jax.experimental import pallas as pl
from jax.experimental.pallas import tpu as pltpu
from jax.experimental.pallas import tpu_sc as plsc

T, ALPHA, BETA = 20, 0.9, 0.8
B, K, IN = 16, 16, 10000
D0, D1, D2, D3 = 5000, 2500, 1250, 625
P0, P1, P2, P3 = 5120, 2560, 1280, 768
NW = 32
N0, N1, N2, N3 = P0 // NW, P1 // NW, P2 // NW, P3 // NW
C0, C1, C2, C3 = N0 * K // 128, N1 * K // 128, N2 * K // 128, N3 * K // 128
JJ = 128 // K


def _mesh():
    return plsc.VectorSubcoreMesh(core_axis_name="c", subcore_axis_name="s")


_CP = pltpu.CompilerParams(use_tc_tiling_on_sc=False)


def _wid():
    return lax.axis_index("c") * 16 + lax.axis_index("s")


def _spiking_kernel(N, C, NBUF, stride_out):
    def body(tbl_h, idx_h, wb_h, b_h, th_h, out_h, *scr):
        wb_v, b_v, th_v, idx_v = scr[0], scr[1], scr[2], scr[3]
        xg = scr[4:4 + NBUF]
        syn_v, mem_v, h_v = scr[4 + NBUF:7 + NBUF]
        sem = scr[7 + NBUF:7 + 2 * NBUF]
        w = _wid()
        pltpu.sync_copy(wb_h.at[pl.ds(w * (N * K), N * K)], wb_v)
        pltpu.sync_copy(b_h.at[pl.ds(w * N, N)], b_v)
        pltpu.sync_copy(th_h.at[pl.ds(w * N, N)], th_v)

        @pl.loop(0, N)
        def _(j):
            z = jnp.zeros((B,), jnp.float32)
            syn_v[j] = z
            mem_v[j] = z

        @pl.loop(0, T)
        def _(t):
            pltpu.sync_copy(idx_h.at[w, t], idx_v)
            for bi in range(NBUF):
                pltpu.async_copy(tbl_h.at[idx_v.at[bi]], xg[bi], sem[bi])

            @pl.loop(0, C, step=NBUF)
            def _(c0):
                for bi in range(NBUF):
                    c = c0 + bi
                    pltpu.make_async_copy(
                        tbl_h.at[idx_v.at[c]], xg[bi], sem[bi]).wait()
                    for jj in range(JJ):
                        j = c * JJ + jj
                        acc = b_v[j]
                        for k in range(K):
                            acc = acc + wb_v[j * K + k] * xg[bi][jj * K + k]
                        th = th_v[j]
                        old_mem = mem_v[j]
                        reset = jnp.where(old_mem - th > 0, th, 0.0)
                        syn = ALPHA * syn_v[j] + acc
                        mem = BETA * old_mem + syn - reset
                        syn_v[j] = syn
                        mem_v[j] = mem
                        h_v[j] = jnp.where(mem - th > 0, 1.0, 0.0)

                    @pl.when(c + NBUF < C)
                    def _():
                        pltpu.async_copy(
                            tbl_h.at[idx_v.at[c + NBUF]], xg[bi], sem[bi])

            if stride_out is not None:
                pltpu.sync_copy(h_v, out_h.at[pl.ds(t * stride_out + w * N, N)])

        if stride_out is None:
            pltpu.sync_copy(mem_v, out_h.at[pl.ds(w * N, N)])

    scratch = (
        [pltpu.VMEM((N * K, B), jnp.float32),
         pltpu.VMEM((N, B), jnp.float32),
         pltpu.VMEM((N, B), jnp.float32),
         pltpu.VMEM((C, 128), jnp.int32)]
        + [pltpu.VMEM((128, B), jnp.float32)] * NBUF
        + [pltpu.VMEM((N, B), jnp.float32)] * 3
        + [pltpu.SemaphoreType.DMA] * NBUF
    )
    return body, scratch


def _relu_kernel(N, C):
    def body(tbl_h, idx_h, wb_h, b_h, out_h, xg_v, wb_v, b_v, idx_v, o_v, sem):
        w = _wid()
        pltpu.sync_copy(idx_h.at[w], idx_v)
        handles = [
            pltpu.async_copy(tbl_h.at[idx_v.at[c]],
                             xg_v.at[pl.ds(c * 128, 128)], sem)
            for c in range(C)
        ]
        pltpu.sync_copy(wb_h.at[pl.ds(w * (N * K), N * K)], wb_v)
        pltpu.sync_copy(b_h.at[pl.ds(w * N, N)], b_v)
        for h in handles:
            h.wait()

        @pl.loop(0, N)
        def _(j):
            acc = b_v[j]
            for k in range(K):
                acc = acc + wb_v[j * K + k] * xg_v[j * K + k]
            o_v[j] = jnp.maximum(acc, 0.0)

        pltpu.sync_copy(o_v, out_h.at[pl.ds(w * N, N)])

    return body


def _fc_kernel(x3_h, fcw_h, fcb_h, out_h, x3_v, fcw_v, acc_v, sem):
    w = _wid()

    @pl.when(w == 0)
    def _():
        pltpu.sync_copy(x3_h, x3_v)
        pltpu.sync_copy(fcw_h, fcw_v)
        pltpu.sync_copy(fcb_h, acc_v)
        for o in range(2):
            @pl.loop(0, P3)
            def _(d):
                acc_v[o] = acc_v[o] + fcw_v[o * P3 + d] * x3_v[d]
        pltpu.sync_copy(acc_v, out_h)


def _pad_rows(a, P):
    pad = P - a.shape[0]
    if pad == 0:
        return a
    return jnp.concatenate([a, jnp.zeros((pad,) + a.shape[1:], a.dtype)], axis=0)


def _prep(knn, W, bvec, P, stride_t=None):
    knnp = _pad_rows(knn.astype(jnp.int32), P)
    Wp = _pad_rows(W, P)
    flat = knnp.reshape(-1)
    if stride_t is not None:
        idx = flat[None, :] + (jnp.arange(T, dtype=jnp.int32) * stride_t)[:, None]
        idx = idx.reshape(T, NW, -1, 128).transpose(1, 0, 2, 3)
    else:
        idx = flat.reshape(NW, -1, 128)
    wb = jnp.broadcast_to(Wp.reshape(-1, 1), (P * K, B)).astype(jnp.float32)
    bb = jnp.broadcast_to(_pad_rows(bvec.reshape(-1, 1), P), (P, B)).astype(jnp.float32)
    return idx, wb, bb


def kernel(input, W0, b0, W1, b1, W2, b2, W3, b3, knn0, knn1, knn2, knn3,
           th0, th1, fcW, fcb):
    f32 = jnp.float32
    xT = input.transpose(1, 2, 0).reshape(T * IN, B)

    idx0, wb0, b0b = _prep(knn0, W0, b0, P0, stride_t=IN)
    idx1, wb1, b1b = _prep(knn1, W1, b1, P1, stride_t=P0)
    idx2, wb2, b2b = _prep(knn2, W2, b2, P2)
    idx3, wb3, b3b = _prep(knn3, W3, b3, P3)
    th0b = jnp.broadcast_to(_pad_rows(th0.reshape(-1, 1), P0), (P0, B)).astype(f32)
    th1b = jnp.broadcast_to(_pad_rows(th1.reshape(-1, 1), P1), (P1, B)).astype(f32)
    fcWb = jnp.broadcast_to(
        _pad_rows(fcW.T, P3).T.reshape(-1, 1), (2 * P3, B)).astype(f32)
    fcbb = jnp.broadcast_to(fcb.reshape(-1, 1), (2, B)).astype(f32)

    bodyA, scrA = _spiking_kernel(N0, C0, 4, P0)
    kA = pl.kernel(bodyA, compiler_params=_CP, mesh=_mesh(),
                   out_type=jax.ShapeDtypeStruct((T * P0, B), f32),
                   scratch_types=scrA)
    h0 = kA(xT, idx0, wb0, b0b, th0b)

    bodyB, scrB = _spiking_kernel(N1, C1, 5, None)
    kB = pl.kernel(bodyB, compiler_params=_CP, mesh=_mesh(),
                   out_type=jax.ShapeDtypeStruct((P1, B), f32),
                   scratch_types=scrB)
    m1 = kB(h0, idx1, wb1, b1b, th1b)

    kC = functools.partial(
        pl.kernel, compiler_params=_CP, out_type=jax.ShapeDtypeStruct((P2, B), f32), mesh=_mesh(),
        scratch_types=[
            pltpu.VMEM((N2 * K, B), f32), pltpu.VMEM((N2 * K, B), f32),
            pltpu.VMEM((N2, B), f32), pltpu.VMEM((C2, 128), jnp.int32),
            pltpu.VMEM((N2, B), f32), pltpu.SemaphoreType.DMA,
        ])(_relu_kernel(N2, C2))
    x2 = kC(m1, idx2, wb2, b2b)

    kD = functools.partial(
        pl.kernel, compiler_params=_CP, out_type=jax.ShapeDtypeStruct((P3, B), f32), mesh=_mesh(),
        scratch_types=[
            pltpu.VMEM((N3 * K, B), f32), pltpu.VMEM((N3 * K, B), f32),
            pltpu.VMEM((N3, B), f32), pltpu.VMEM((C3, 128), jnp.int32),
            pltpu.VMEM((N3, B), f32), pltpu.SemaphoreType.DMA,
        ])(_relu_kernel(N3, C3))
    x3 = kD(x2, idx3, wb3, b3b)

    kE = functools.partial(
        pl.kernel, compiler_params=_CP, out_type=jax.ShapeDtypeStruct((2, B), f32), mesh=_mesh(),
        scratch_types=[
            pltpu.VMEM((P3, B), f32), pltpu.VMEM((2 * P3, B), f32),
            pltpu.VMEM((2, B), f32), pltpu.SemaphoreType.DMA,
        ])(_fc_kernel)
    angle = kE(x3, fcWb, fcbb)

    return angle.T


# speedup vs baseline: 8.3101x; 1.1369x over previous
import functools

import jax
import jax.numpy as jnp
from jax import lax
from jax.experimental import pallas as pl
from jax.experimental.pallas import tpu as pltpu
from jax.experimental.pallas import tpu_sc as plsc

T, ALPHA, BETA = 20, 0.9, 0.8
B, K, IN = 16, 16, 10000
D0, D1, D2, D3 = 5000, 2500, 1250, 625
P0, P1, P2, P3 = 5120, 2560, 1280, 768
NW = 32
N0, N1, N2, N3 = P0 // NW, P1 // NW, P2 // NW, P3 // NW
C0, C1, C2, C3 = N0 * K // 128, N1 * K // 128, N2 * K // 128, N3 * K // 128
JJ = 128 // K


def _mesh():
    return plsc.VectorSubcoreMesh(core_axis_name="c", subcore_axis_name="s")


_CP = pltpu.CompilerParams(use_tc_tiling_on_sc=False)


def _wid():
    return lax.axis_index("c") * 16 + lax.axis_index("s")


_GDN = lax.GatherDimensionNumbers(
    offset_dims=(), collapsed_slice_dims=(0,), start_index_map=(0,))


def _lane(wv, k):
    return lax.gather(wv, jnp.full((B, 1), k, jnp.int32), _GDN, (1,),
                      mode=lax.GatherScatterMode.PROMISE_IN_BOUNDS)


def _wsum(wv, xg_at, init):
    parts = [init, None, None, None]
    for k in range(K):
        t = _lane(wv, k) * xg_at(k)
        p = k % 4
        parts[p] = t if parts[p] is None else parts[p] + t
    return (parts[0] + parts[1]) + (parts[2] + parts[3])


def _spiking_kernel(N, C, NBUF, stride_out):
    def body(tbl_h, idx_h, wp_h, b_h, th_h, out_h, *scr):
        wp_v, b_v, th_v, idx_v = scr[0], scr[1], scr[2], scr[3]
        xg = scr[4:4 + NBUF]
        syn_v, mem_v, h_v = scr[4 + NBUF:7 + NBUF]
        sem = scr[7 + NBUF:7 + 2 * NBUF]
        w = _wid()
        pltpu.sync_copy(wp_h.at[pl.ds(w * N, N)], wp_v)
        pltpu.sync_copy(b_h.at[pl.ds(w * N, N)], b_v)
        pltpu.sync_copy(th_h.at[pl.ds(w * N, N)], th_v)

        @pl.loop(0, N)
        def _(j):
            z = jnp.zeros((B,), jnp.float32)
            syn_v[j] = z
            mem_v[j] = z

        @pl.loop(0, T)
        def _(t):
            pltpu.sync_copy(idx_h.at[w, t], idx_v)
            for bi in range(NBUF):
                pltpu.async_copy(tbl_h.at[idx_v.at[bi]], xg[bi], sem[bi])

            @pl.loop(0, C, step=NBUF)
            def _(c0):
                for bi in range(NBUF):
                    c = c0 + bi
                    pltpu.make_async_copy(
                        tbl_h.at[idx_v.at[c]], xg[bi], sem[bi]).wait()
                    for jj in range(JJ):
                        j = c * JJ + jj
                        wv = wp_v[j]
                        acc = _wsum(wv, lambda k, _b=bi, _j=jj: xg[_b][_j * K + k],
                                    b_v[j])
                        th = th_v[j]
                        old_mem = mem_v[j]
                        reset = jnp.where(old_mem - th > 0, th, 0.0)
                        syn = ALPHA * syn_v[j] + acc
                        mem = BETA * old_mem + syn - reset
                        syn_v[j] = syn
                        mem_v[j] = mem
                        h_v[j] = jnp.where(mem - th > 0, 1.0, 0.0)

                    @pl.when(c + NBUF < C)
                    def _():
                        pltpu.async_copy(
                            tbl_h.at[idx_v.at[c + NBUF]], xg[bi], sem[bi])

            if stride_out is not None:
                pltpu.sync_copy(h_v, out_h.at[pl.ds(t * stride_out + w * N, N)])

        if stride_out is None:
            pltpu.sync_copy(mem_v, out_h.at[pl.ds(w * N, N)])

    scratch = (
        [pltpu.VMEM((N, K), jnp.float32),
         pltpu.VMEM((N, B), jnp.float32),
         pltpu.VMEM((N, B), jnp.float32),
         pltpu.VMEM((C, 128), jnp.int32)]
        + [pltpu.VMEM((128, B), jnp.float32)] * NBUF
        + [pltpu.VMEM((N, B), jnp.float32)] * 3
        + [pltpu.SemaphoreType.DMA] * NBUF
    )
    return body, scratch


def _relu_kernel(N, C):
    def body(tbl_h, idx_h, wp_h, b_h, out_h, xg_v, wp_v, b_v, idx_v, o_v, sem):
        w = _wid()
        pltpu.sync_copy(idx_h.at[w], idx_v)
        handles = [
            pltpu.async_copy(tbl_h.at[idx_v.at[c]],
                             xg_v.at[pl.ds(c * 128, 128)], sem)
            for c in range(C)
        ]
        pltpu.sync_copy(wp_h.at[pl.ds(w * N, N)], wp_v)
        pltpu.sync_copy(b_h.at[pl.ds(w * N, N)], b_v)
        for h in handles:
            h.wait()

        @pl.loop(0, N)
        def _(j):
            acc = _wsum(wp_v[j], lambda k: xg_v[j * K + k], b_v[j])
            o_v[j] = jnp.maximum(acc, 0.0)

        pltpu.sync_copy(o_v, out_h.at[pl.ds(w * N, N)])

    return body


def _fc_kernel(x3_h, fcw_h, fcb_h, out_h, x3_v, fcw_v, acc_v, sem):
    w = _wid()
    G = P3 // B

    @pl.when(w == 0)
    def _():
        pltpu.sync_copy(x3_h, x3_v)
        pltpu.sync_copy(fcw_h, fcw_v)
        pltpu.sync_copy(fcb_h, acc_v)
        for o in range(2):
            @pl.loop(0, G)
            def _(g):
                acc_v[o] = _wsum(fcw_v[o * G + g],
                                 lambda k, _g=g: x3_v[_g * B + k], acc_v[o])
        pltpu.sync_copy(acc_v, out_h)


def _pad_rows(a, P):
    pad = P - a.shape[0]
    if pad == 0:
        return a
    return jnp.concatenate([a, jnp.zeros((pad,) + a.shape[1:], a.dtype)], axis=0)


def _prep(knn, W, bvec, P, stride_t=None):
    knnp = _pad_rows(knn.astype(jnp.int32), P)
    Wp = _pad_rows(W, P)
    flat = knnp.reshape(-1)
    if stride_t is not None:
        idx = flat[None, :] + (jnp.arange(T, dtype=jnp.int32) * stride_t)[:, None]
        idx = idx.reshape(T, NW, -1, 128).transpose(1, 0, 2, 3)
    else:
        idx = flat.reshape(NW, -1, 128)
    bb = jnp.broadcast_to(_pad_rows(bvec.reshape(-1, 1), P), (P, B)).astype(jnp.float32)
    return idx, Wp.astype(jnp.float32), bb


def kernel(input, W0, b0, W1, b1, W2, b2, W3, b3, knn0, knn1, knn2, knn3,
           th0, th1, fcW, fcb):
    f32 = jnp.float32
    xT = input.transpose(1, 2, 0).reshape(T * IN, B)

    idx0, wb0, b0b = _prep(knn0, W0, b0, P0, stride_t=IN)
    idx1, wb1, b1b = _prep(knn1, W1, b1, P1, stride_t=P0)
    idx2, wb2, b2b = _prep(knn2, W2, b2, P2)
    idx3, wb3, b3b = _prep(knn3, W3, b3, P3)
    th0b = jnp.broadcast_to(_pad_rows(th0.reshape(-1, 1), P0), (P0, B)).astype(f32)
    th1b = jnp.broadcast_to(_pad_rows(th1.reshape(-1, 1), P1), (P1, B)).astype(f32)
    fcWb = _pad_rows(fcW.T, P3).T.reshape(2 * P3 // B, B).astype(f32)
    fcbb = jnp.broadcast_to(fcb.reshape(-1, 1), (2, B)).astype(f32)

    bodyA, scrA = _spiking_kernel(N0, C0, 4, P0)
    kA = pl.kernel(bodyA, compiler_params=_CP, mesh=_mesh(),
                   out_type=jax.ShapeDtypeStruct((T * P0, B), f32),
                   scratch_types=scrA)
    h0 = kA(xT, idx0, wb0, b0b, th0b)

    bodyB, scrB = _spiking_kernel(N1, C1, 5, None)
    kB = pl.kernel(bodyB, compiler_params=_CP, mesh=_mesh(),
                   out_type=jax.ShapeDtypeStruct((P1, B), f32),
                   scratch_types=scrB)
    m1 = kB(h0, idx1, wb1, b1b, th1b)

    kC = functools.partial(
        pl.kernel, compiler_params=_CP, out_type=jax.ShapeDtypeStruct((P2, B), f32), mesh=_mesh(),
        scratch_types=[
            pltpu.VMEM((N2 * K, B), f32), pltpu.VMEM((N2, K), f32),
            pltpu.VMEM((N2, B), f32), pltpu.VMEM((C2, 128), jnp.int32),
            pltpu.VMEM((N2, B), f32), pltpu.SemaphoreType.DMA,
        ])(_relu_kernel(N2, C2))
    x2 = kC(m1, idx2, wb2, b2b)

    kD = functools.partial(
        pl.kernel, compiler_params=_CP, out_type=jax.ShapeDtypeStruct((P3, B), f32), mesh=_mesh(),
        scratch_types=[
            pltpu.VMEM((N3 * K, B), f32), pltpu.VMEM((N3, K), f32),
            pltpu.VMEM((N3, B), f32), pltpu.VMEM((C3, 128), jnp.int32),
            pltpu.VMEM((N3, B), f32), pltpu.SemaphoreType.DMA,
        ])(_relu_kernel(N3, C3))
    x3 = kD(x2, idx3, wb3, b3b)

    kE = functools.partial(
        pl.kernel, compiler_params=_CP, out_type=jax.ShapeDtypeStruct((2, B), f32), mesh=_mesh(),
        scratch_types=[
            pltpu.VMEM((P3, B), f32), pltpu.VMEM((2 * P3 // B, B), f32),
            pltpu.VMEM((2, B), f32), pltpu.SemaphoreType.DMA,
        ])(_fc_kernel)
    angle = kE(x3, fcWb, fcbb)

    return angle.T


# speedup vs baseline: 9.6704x; 1.1637x over previous
import functools

import jax
import jax.numpy as jnp
from jax import lax
from jax.experimental import pallas as pl
from jax.experimental.pallas import tpu as pltpu
from jax.experimental.pallas import tpu_sc as plsc

T, ALPHA, BETA = 20, 0.9, 0.8
B, K, IN = 16, 16, 10000
D0, D1, D2, D3 = 5000, 2500, 1250, 625
P0, P1, P2, P3 = 5120, 2560, 1280, 768
NW = 32
N0, N1, N2, N3 = P0 // NW, P1 // NW, P2 // NW, P3 // NW
C0, C1, C2, C3 = N0 * K // 128, N1 * K // 128, N2 * K // 128, N3 * K // 128
JJ = 128 // K


def _mesh():
    return plsc.VectorSubcoreMesh(core_axis_name="c", subcore_axis_name="s")


_CP = pltpu.CompilerParams(use_tc_tiling_on_sc=False)


def _wid():
    return lax.axis_index("c") * 16 + lax.axis_index("s")


_GDN = lax.GatherDimensionNumbers(
    offset_dims=(), collapsed_slice_dims=(0,), start_index_map=(0,))


def _lane(wv, k):
    return lax.gather(wv, jnp.full((B, 1), k, jnp.int32), _GDN, (1,),
                      mode=lax.GatherScatterMode.PROMISE_IN_BOUNDS)


def _wsum(wv, xg_at, init):
    parts = [init, None, None, None]
    for k in range(K):
        t = _lane(wv, k) * xg_at(k)
        p = k % 4
        parts[p] = t if parts[p] is None else parts[p] + t
    return (parts[0] + parts[1]) + (parts[2] + parts[3])


def _spiking_kernel(N, C, stride_out):
    def body(tbl_h, idx_h, wp_h, b_h, th_h, out_h, *scr):
        wp_v, b_v, th_v = scr[0], scr[1], scr[2]
        idx = scr[3:5]
        xg = scr[5:7]
        syn_v, mem_v, h_v = scr[7], scr[8], scr[9]
        sem = scr[10:12]
        w = _wid()
        pltpu.sync_copy(wp_h.at[pl.ds(w * N, N)], wp_v)
        pltpu.sync_copy(b_h.at[pl.ds(w * N, N)], b_v)
        pltpu.sync_copy(th_h.at[pl.ds(w * N, N)], th_v)

        @pl.loop(0, N)
        def _(j):
            z = jnp.zeros((B,), jnp.float32)
            syn_v[j] = z
            mem_v[j] = z

        def fire(t, bi):
            pltpu.sync_copy(idx_h.at[w, t], idx[bi])
            for c in range(C):
                pltpu.async_copy(tbl_h.at[idx[bi].at[c]],
                                 xg[bi].at[pl.ds(c * 128, 128)], sem[bi])

        def drain(bi):
            for c in range(C):
                pltpu.make_async_copy(tbl_h.at[idx[bi].at[c]],
                                      xg[bi].at[pl.ds(c * 128, 128)],
                                      sem[bi]).wait()

        fire(0, 0)
        for t in range(T):
            cur = t % 2
            if t + 1 < T:
                fire(t + 1, 1 - cur)
            drain(cur)
            xgc = xg[cur]

            @pl.loop(0, N)
            def _(j, _x=xgc):
                acc = _wsum(wp_v[j], lambda k, _j=j: _x[_j * K + k], b_v[j])
                th = th_v[j]
                old_mem = mem_v[j]
                reset = jnp.where(old_mem - th > 0, th, 0.0)
                syn = ALPHA * syn_v[j] + acc
                mem = BETA * old_mem + syn - reset
                syn_v[j] = syn
                mem_v[j] = mem
                h_v[j] = jnp.where(mem - th > 0, 1.0, 0.0)

            if stride_out is not None:
                pltpu.sync_copy(h_v, out_h.at[pl.ds(t * stride_out + w * N, N)])

        if stride_out is None:
            pltpu.sync_copy(mem_v, out_h.at[pl.ds(w * N, N)])

    scratch = (
        [pltpu.VMEM((N, K), jnp.float32),
         pltpu.VMEM((N, B), jnp.float32),
         pltpu.VMEM((N, B), jnp.float32)]
        + [pltpu.VMEM((C, 128), jnp.int32)] * 2
        + [pltpu.VMEM((N * K, B), jnp.float32)] * 2
        + [pltpu.VMEM((N, B), jnp.float32)] * 3
        + [pltpu.SemaphoreType.DMA] * 2
    )
    return body, scratch


def _relu_kernel(N, C):
    def body(tbl_h, idx_h, wp_h, b_h, out_h, xg_v, wp_v, b_v, idx_v, o_v, sem):
        w = _wid()
        pltpu.sync_copy(idx_h.at[w], idx_v)
        handles = [
            pltpu.async_copy(tbl_h.at[idx_v.at[c]],
                             xg_v.at[pl.ds(c * 128, 128)], sem)
            for c in range(C)
        ]
        pltpu.sync_copy(wp_h.at[pl.ds(w * N, N)], wp_v)
        pltpu.sync_copy(b_h.at[pl.ds(w * N, N)], b_v)
        for h in handles:
            h.wait()

        @pl.loop(0, N)
        def _(j):
            acc = _wsum(wp_v[j], lambda k: xg_v[j * K + k], b_v[j])
            o_v[j] = jnp.maximum(acc, 0.0)

        pltpu.sync_copy(o_v, out_h.at[pl.ds(w * N, N)])

    return body


def _fc_kernel(x3_h, fcw_h, fcb_h, out_h, x3_v, fcw_v, acc_v, sem):
    w = _wid()
    G = P3 // B

    @pl.when(w == 0)
    def _():
        pltpu.sync_copy(x3_h, x3_v)
        pltpu.sync_copy(fcw_h, fcw_v)
        pltpu.sync_copy(fcb_h, acc_v)
        for o in range(2):
            @pl.loop(0, G)
            def _(g):
                acc_v[o] = _wsum(fcw_v[o * G + g],
                                 lambda k, _g=g: x3_v[_g * B + k], acc_v[o])
        pltpu.sync_copy(acc_v, out_h)


def _pad_rows(a, P):
    pad = P - a.shape[0]
    if pad == 0:
        return a
    return jnp.concatenate([a, jnp.zeros((pad,) + a.shape[1:], a.dtype)], axis=0)


def _prep(knn, W, bvec, P, stride_t=None):
    knnp = _pad_rows(knn.astype(jnp.int32), P)
    Wp = _pad_rows(W, P)
    flat = knnp.reshape(-1)
    if stride_t is not None:
        idx = flat[None, :] + (jnp.arange(T, dtype=jnp.int32) * stride_t)[:, None]
        idx = idx.reshape(T, NW, -1, 128).transpose(1, 0, 2, 3)
    else:
        idx = flat.reshape(NW, -1, 128)
    bb = jnp.broadcast_to(_pad_rows(bvec.reshape(-1, 1), P), (P, B)).astype(jnp.float32)
    return idx, Wp.astype(jnp.float32), bb


def kernel(input, W0, b0, W1, b1, W2, b2, W3, b3, knn0, knn1, knn2, knn3,
           th0, th1, fcW, fcb):
    f32 = jnp.float32
    xT = input.transpose(1, 2, 0).reshape(T * IN, B)

    idx0, wb0, b0b = _prep(knn0, W0, b0, P0, stride_t=IN)
    idx1, wb1, b1b = _prep(knn1, W1, b1, P1, stride_t=P0)
    idx2, wb2, b2b = _prep(knn2, W2, b2, P2)
    idx3, wb3, b3b = _prep(knn3, W3, b3, P3)
    th0b = jnp.broadcast_to(_pad_rows(th0.reshape(-1, 1), P0), (P0, B)).astype(f32)
    th1b = jnp.broadcast_to(_pad_rows(th1.reshape(-1, 1), P1), (P1, B)).astype(f32)
    fcWb = _pad_rows(fcW.T, P3).T.reshape(2 * P3 // B, B).astype(f32)
    fcbb = jnp.broadcast_to(fcb.reshape(-1, 1), (2, B)).astype(f32)

    bodyA, scrA = _spiking_kernel(N0, C0, P0)
    kA = pl.kernel(bodyA, compiler_params=_CP, mesh=_mesh(),
                   out_type=jax.ShapeDtypeStruct((T * P0, B), f32),
                   scratch_types=scrA)
    h0 = kA(xT, idx0, wb0, b0b, th0b)

    bodyB, scrB = _spiking_kernel(N1, C1, None)
    kB = pl.kernel(bodyB, compiler_params=_CP, mesh=_mesh(),
                   out_type=jax.ShapeDtypeStruct((P1, B), f32),
                   scratch_types=scrB)
    m1 = kB(h0, idx1, wb1, b1b, th1b)

    kC = functools.partial(
        pl.kernel, compiler_params=_CP, out_type=jax.ShapeDtypeStruct((P2, B), f32), mesh=_mesh(),
        scratch_types=[
            pltpu.VMEM((N2 * K, B), f32), pltpu.VMEM((N2, K), f32),
            pltpu.VMEM((N2, B), f32), pltpu.VMEM((C2, 128), jnp.int32),
            pltpu.VMEM((N2, B), f32), pltpu.SemaphoreType.DMA,
        ])(_relu_kernel(N2, C2))
    x2 = kC(m1, idx2, wb2, b2b)

    kD = functools.partial(
        pl.kernel, compiler_params=_CP, out_type=jax.ShapeDtypeStruct((P3, B), f32), mesh=_mesh(),
        scratch_types=[
            pltpu.VMEM((N3 * K, B), f32), pltpu.VMEM((N3, K), f32),
            pltpu.VMEM((N3, B), f32), pltpu.VMEM((C3, 128), jnp.int32),
            pltpu.VMEM((N3, B), f32), pltpu.SemaphoreType.DMA,
        ])(_relu_kernel(N3, C3))
    x3 = kD(x2, idx3, wb3, b3b)

    kE = functools.partial(
        pl.kernel, compiler_params=_CP, out_type=jax.ShapeDtypeStruct((2, B), f32), mesh=_mesh(),
        scratch_types=[
            pltpu.VMEM((P3, B), f32), pltpu.VMEM((2 * P3 // B, B), f32),
            pltpu.VMEM((2, B), f32), pltpu.SemaphoreType.DMA,
        ])(_fc_kernel)
    angle = kE(x3, fcWb, fcbb)

    return angle.T


# speedup vs baseline: 12.5721x; 1.3001x over previous
import functools

import jax
import jax.numpy as jnp
from jax import lax
from jax.experimental import pallas as pl
from jax.experimental.pallas import tpu as pltpu
from jax.experimental.pallas import tpu_sc as plsc

T, ALPHA, BETA = 20, 0.9, 0.8
B, K, IN = 16, 16, 10000
D0, D1, D2, D3 = 5000, 2500, 1250, 625
P0, P1, P2, P3 = 5120, 2560, 1280, 768
NW = 32
N0, N1, N2, N3 = P0 // NW, P1 // NW, P2 // NW, P3 // NW
C0, C1, C2, C3 = N0 * K // 128, N1 * K // 128, N2 * K // 128, N3 * K // 128
JJ = 128 // K


def _mesh():
    return plsc.VectorSubcoreMesh(core_axis_name="c", subcore_axis_name="s")


_CP = pltpu.CompilerParams(use_tc_tiling_on_sc=False)


def _wid():
    return lax.axis_index("c") * 16 + lax.axis_index("s")


_GDN = lax.GatherDimensionNumbers(
    offset_dims=(), collapsed_slice_dims=(0,), start_index_map=(0,))


def _lane(wv, k):
    return lax.gather(wv, jnp.full((B, 1), k, jnp.int32), _GDN, (1,),
                      mode=lax.GatherScatterMode.PROMISE_IN_BOUNDS)


def _wsum(wv, xg_at, init):
    parts = [init, None, None, None]
    for k in range(K):
        t = _lane(wv, k) * xg_at(k)
        p = k % 4
        parts[p] = t if parts[p] is None else parts[p] + t
    return (parts[0] + parts[1]) + (parts[2] + parts[3])


def _spiking_kernel(N, C, stride_out):
    def body(tbl_h, idx_h, wp_h, b_h, th_h, out_h, *scr):
        wp_v, b_v, th_v, idx_v = scr[0], scr[1], scr[2], scr[3]
        xg = scr[4:6]
        syn_v, mem_v, hst_v = scr[6], scr[7], scr[8]
        sem = scr[9:11]
        w = _wid()
        pltpu.sync_copy(wp_h.at[pl.ds(w * N, N)], wp_v)
        pltpu.sync_copy(b_h.at[pl.ds(w * N, N)], b_v)
        pltpu.sync_copy(th_h.at[pl.ds(w * N, N)], th_v)
        pltpu.sync_copy(idx_h.at[w], idx_v)

        def fire(g, bi):
            pltpu.async_copy(tbl_h.at[idx_v.at[g]], xg[bi], sem[bi])

        def drain(g, bi):
            pltpu.make_async_copy(tbl_h.at[idx_v.at[g]], xg[bi], sem[bi]).wait()

        fire(0, 0)
        for g in range(C):
            cur = g % 2
            if g + 1 < C:
                fire(g + 1, 1 - cur)
            drain(g, cur)
            xgc = xg[cur]
            for jj in range(JJ):
                j = g * JJ + jj
                z = jnp.zeros((B,), jnp.float32)
                syn_v[jj] = z
                mem_v[jj] = z

            @pl.loop(0, T)
            def _(t, _x=xgc, _g=g):
                for jj in range(JJ):
                    j = _g * JJ + jj
                    acc = _wsum(wp_v[j], lambda k, _jj=jj: _x[_jj * K + k, t],
                                b_v[j])
                    th = th_v[j]
                    old_mem = mem_v[jj]
                    reset = jnp.where(old_mem - th > 0, th, 0.0)
                    syn = ALPHA * syn_v[jj] + acc
                    mem = BETA * old_mem + syn - reset
                    syn_v[jj] = syn
                    mem_v[jj] = mem
                    if stride_out is not None:
                        hst_v[jj * T + t] = jnp.where(mem - th > 0, 1.0, 0.0)

            if stride_out is not None:
                pltpu.sync_copy(
                    hst_v, out_h.at[pl.ds((w * N + g * JJ) * T, JJ * T)])
            else:
                for jj in range(JJ):
                    hst_v[g * JJ + jj] = mem_v[jj]

        if stride_out is None:
            pltpu.sync_copy(hst_v, out_h.at[pl.ds(w * N, N)])

    hst_rows = JJ * T if stride_out is not None else N
    scratch = (
        [pltpu.VMEM((N, K), jnp.float32),
         pltpu.VMEM((N, B), jnp.float32),
         pltpu.VMEM((N, B), jnp.float32),
         pltpu.VMEM((C, 128), jnp.int32)]
        + [pltpu.VMEM((128, T, B), jnp.float32)] * 2
        + [pltpu.VMEM((JJ, B), jnp.float32)] * 2
        + [pltpu.VMEM((hst_rows, B), jnp.float32)]
        + [pltpu.SemaphoreType.DMA] * 2
    )
    return body, scratch


def _relu_kernel(N, C):
    def body(tbl_h, idx_h, wp_h, b_h, out_h, xg_v, wp_v, b_v, idx_v, o_v, sem):
        w = _wid()
        pltpu.sync_copy(idx_h.at[w], idx_v)
        handles = [
            pltpu.async_copy(tbl_h.at[idx_v.at[c]],
                             xg_v.at[pl.ds(c * 128, 128)], sem)
            for c in range(C)
        ]
        pltpu.sync_copy(wp_h.at[pl.ds(w * N, N)], wp_v)
        pltpu.sync_copy(b_h.at[pl.ds(w * N, N)], b_v)
        for h in handles:
            h.wait()

        @pl.loop(0, N)
        def _(j):
            acc = _wsum(wp_v[j], lambda k: xg_v[j * K + k], b_v[j])
            o_v[j] = jnp.maximum(acc, 0.0)

        pltpu.sync_copy(o_v, out_h.at[pl.ds(w * N, N)])

    return body


def _fc_kernel(x3_h, fcw_h, fcb_h, out_h, x3_v, fcw_v, acc_v, sem):
    w = _wid()
    G = P3 // B

    @pl.when(w == 0)
    def _():
        pltpu.sync_copy(x3_h, x3_v)
        pltpu.sync_copy(fcw_h, fcw_v)
        pltpu.sync_copy(fcb_h, acc_v)
        for o in range(2):
            @pl.loop(0, G)
            def _(g):
                acc_v[o] = _wsum(fcw_v[o * G + g],
                                 lambda k, _g=g: x3_v[_g * B + k], acc_v[o])
        pltpu.sync_copy(acc_v, out_h)


def _pad_rows(a, P):
    pad = P - a.shape[0]
    if pad == 0:
        return a
    return jnp.concatenate([a, jnp.zeros((pad,) + a.shape[1:], a.dtype)], axis=0)


def _prep(knn, W, bvec, P):
    knnp = _pad_rows(knn.astype(jnp.int32), P)
    Wp = _pad_rows(W, P)
    idx = knnp.reshape(NW, -1, 128)
    bb = jnp.broadcast_to(_pad_rows(bvec.reshape(-1, 1), P), (P, B)).astype(jnp.float32)
    return idx, Wp.astype(jnp.float32), bb


def kernel(input, W0, b0, W1, b1, W2, b2, W3, b3, knn0, knn1, knn2, knn3,
           th0, th1, fcW, fcb):
    f32 = jnp.float32
    xT = input.transpose(2, 1, 0)

    idx0, wb0, b0b = _prep(knn0, W0, b0, P0)
    idx1, wb1, b1b = _prep(knn1, W1, b1, P1)
    idx2, wb2, b2b = _prep(knn2, W2, b2, P2)
    idx3, wb3, b3b = _prep(knn3, W3, b3, P3)
    th0b = jnp.broadcast_to(_pad_rows(th0.reshape(-1, 1), P0), (P0, B)).astype(f32)
    th1b = jnp.broadcast_to(_pad_rows(th1.reshape(-1, 1), P1), (P1, B)).astype(f32)
    fcWb = _pad_rows(fcW.T, P3).T.reshape(2 * P3 // B, B).astype(f32)
    fcbb = jnp.broadcast_to(fcb.reshape(-1, 1), (2, B)).astype(f32)

    bodyA, scrA = _spiking_kernel(N0, C0, P0)
    kA = pl.kernel(bodyA, compiler_params=_CP, mesh=_mesh(),
                   out_type=jax.ShapeDtypeStruct((P0 * T, B), f32),
                   scratch_types=scrA)
    h0 = kA(xT, idx0, wb0, b0b, th0b)

    bodyB, scrB = _spiking_kernel(N1, C1, None)
    kB = pl.kernel(bodyB, compiler_params=_CP, mesh=_mesh(),
                   out_type=jax.ShapeDtypeStruct((P1, B), f32),
                   scratch_types=scrB)
    m1 = kB(h0.reshape(P0, T, B), idx1, wb1, b1b, th1b)

    kC = functools.partial(
        pl.kernel, compiler_params=_CP, out_type=jax.ShapeDtypeStruct((P2, B), f32), mesh=_mesh(),
        scratch_types=[
            pltpu.VMEM((N2 * K, B), f32), pltpu.VMEM((N2, K), f32),
            pltpu.VMEM((N2, B), f32), pltpu.VMEM((C2, 128), jnp.int32),
            pltpu.VMEM((N2, B), f32), pltpu.SemaphoreType.DMA,
        ])(_relu_kernel(N2, C2))
    x2 = kC(m1, idx2, wb2, b2b)

    kD = functools.partial(
        pl.kernel, compiler_params=_CP, out_type=jax.ShapeDtypeStruct((P3, B), f32), mesh=_mesh(),
        scratch_types=[
            pltpu.VMEM((N3 * K, B), f32), pltpu.VMEM((N3, K), f32),
            pltpu.VMEM((N3, B), f32), pltpu.VMEM((C3, 128), jnp.int32),
            pltpu.VMEM((N3, B), f32), pltpu.SemaphoreType.DMA,
        ])(_relu_kernel(N3, C3))
    x3 = kD(x2, idx3, wb3, b3b)

    kE = functools.partial(
        pl.kernel, compiler_params=_CP, out_type=jax.ShapeDtypeStruct((2, B), f32), mesh=_mesh(),
        scratch_types=[
            pltpu.VMEM((P3, B), f32), pltpu.VMEM((2 * P3 // B, B), f32),
            pltpu.VMEM((2, B), f32), pltpu.SemaphoreType.DMA,
        ])(_fc_kernel)
    angle = kE(x3, fcWb, fcbb)

    return angle.T
